# Initial kernel scaffold; baseline (speedup 1.0000x reference)
import jax
import jax.numpy as jnp
from jax.experimental import pallas as pl


def kernel(p, q, x, y, W1, b1, g1, be1, W2, b2, g2, be2):
    raise NotImplementedError("write your pallas kernel here")


# speedup vs baseline: 18.3374x; 18.3374x over previous
import functools
import jax
import jax.numpy as jnp
from jax.experimental import pallas as pl
from jax.experimental.pallas import tpu as pltpu

K = 3
BM = 512


def _knn_l1_kernel(pT_ref, q_ref, x_ref, y_ref, W1_ref, b1_ref,
                   h1_ref, s1_ref, ss1_ref, *, n_src, interp_dtype):
    b = pl.program_id(0)
    j = pl.program_id(1)

    q_blk = q_ref[0]
    d = jnp.zeros((BM, n_src), jnp.float32)
    for c in range(3):
        dq = q_blk[:, c:c + 1]
        dp = pT_ref[0, c:c + 1, :]
        d = d + (dq - dp) ** 2

    iota = jax.lax.broadcasted_iota(jnp.int32, (BM, n_src), 1)
    ws = []
    idxs = []
    for _ in range(K):
        mn = jnp.min(d, axis=1, keepdims=True)
        ik = jnp.min(jnp.where(d == mn, iota, n_src), axis=1,
                     keepdims=True)
        d = jnp.where(iota == ik, jnp.inf, d)
        ws.append(1.0 / jnp.maximum(mn, 1e-10))
        idxs.append(ik)
    wsum = ws[0] + ws[1] + ws[2]

    ST = jnp.zeros((BM, n_src), jnp.float32)
    for k in range(K):
        ST = ST + jnp.where(iota == idxs[k], ws[k] / wsum, 0.0)

    x_b = x_ref[0]
    xi = jax.lax.dot_general(
        x_b.astype(interp_dtype), ST.astype(interp_dtype),
        dimension_numbers=(((1,), (1,)), ((), ())),
        preferred_element_type=jnp.float32)

    W1 = W1_ref[...]
    Cx = x_b.shape[0]
    Wx = W1[:, :Cx].astype(interp_dtype)
    Wy = W1[:, Cx:].astype(interp_dtype)
    y_blk = y_ref[0].astype(interp_dtype)
    h1 = (jax.lax.dot_general(Wx, xi.astype(interp_dtype),
                              dimension_numbers=(((1,), (0,)), ((), ())),
                              preferred_element_type=jnp.float32)
          + jax.lax.dot_general(Wy, y_blk,
                                dimension_numbers=(((1,), (0,)), ((), ())),
                                preferred_element_type=jnp.float32)
          + b1_ref[...])
    h1_ref[0] = h1

    @pl.when(jnp.logical_and(b == 0, j == 0))
    def _():
        s1_ref[...] = jnp.zeros_like(s1_ref)
        ss1_ref[...] = jnp.zeros_like(ss1_ref)

    s1_ref[...] += jnp.sum(h1, axis=1, keepdims=True)
    ss1_ref[...] += jnp.sum(h1 * h1, axis=1, keepdims=True)


def _bn_l2_kernel(h1_ref, s1_ref, ss1_ref, g1_ref, be1_ref, W2_ref, b2_ref,
                  h2_ref, s2_ref, ss2_ref, *, count, interp_dtype):
    b = pl.program_id(0)
    j = pl.program_id(1)

    mean = s1_ref[...] / count
    var = ss1_ref[...] / count - mean * mean
    rstd = jax.lax.rsqrt(var + 1e-5)
    scale = g1_ref[...] * rstd
    shift = be1_ref[...] - mean * scale

    h1 = jnp.maximum(h1_ref[0] * scale + shift, 0.0)
    h2 = (jax.lax.dot_general(W2_ref[...].astype(interp_dtype),
                              h1.astype(interp_dtype),
                              dimension_numbers=(((1,), (0,)), ((), ())),
                              preferred_element_type=jnp.float32)
          + b2_ref[...])
    h2_ref[0] = h2

    @pl.when(jnp.logical_and(b == 0, j == 0))
    def _():
        s2_ref[...] = jnp.zeros_like(s2_ref)
        ss2_ref[...] = jnp.zeros_like(ss2_ref)

    s2_ref[...] += jnp.sum(h2, axis=1, keepdims=True)
    ss2_ref[...] += jnp.sum(h2 * h2, axis=1, keepdims=True)


def _bn_out_kernel(h2_ref, s2_ref, ss2_ref, g2_ref, be2_ref, out_ref, *,
                   count):
    mean = s2_ref[...] / count
    var = ss2_ref[...] / count - mean * mean
    rstd = jax.lax.rsqrt(var + 1e-5)
    scale = g2_ref[...] * rstd
    shift = be2_ref[...] - mean * scale
    out_ref[0] = jnp.maximum(h2_ref[0] * scale + shift, 0.0)


def kernel(p, q, x, y, W1, b1, g1, be1, W2, b2, g2, be2):
    B, N, _ = p.shape
    M = q.shape[1]
    Cx = x.shape[1]
    Cy = y.shape[1]
    C1 = W1.shape[0]
    C2 = W2.shape[0]
    grid = (B, M // BM)
    count = float(B * M)
    interp_dtype = jnp.float32

    pT = jnp.swapaxes(p, 1, 2)
    col = lambda v: v.reshape(-1, 1)

    h1_pre, s1, ss1 = pl.pallas_call(
        functools.partial(_knn_l1_kernel, n_src=N, interp_dtype=interp_dtype),
        grid=grid,
        in_specs=[
            pl.BlockSpec((1, 3, N), lambda b, j: (b, 0, 0)),
            pl.BlockSpec((1, BM, 3), lambda b, j: (b, j, 0)),
            pl.BlockSpec((1, Cx, N), lambda b, j: (b, 0, 0)),
            pl.BlockSpec((1, Cy, BM), lambda b, j: (b, 0, j)),
            pl.BlockSpec((C1, Cx + Cy), lambda b, j: (0, 0)),
            pl.BlockSpec((C1, 1), lambda b, j: (0, 0)),
        ],
        out_specs=[
            pl.BlockSpec((1, C1, BM), lambda b, j: (b, 0, j)),
            pl.BlockSpec((C1, 1), lambda b, j: (0, 0)),
            pl.BlockSpec((C1, 1), lambda b, j: (0, 0)),
        ],
        out_shape=[
            jax.ShapeDtypeStruct((B, C1, M), jnp.float32),
            jax.ShapeDtypeStruct((C1, 1), jnp.float32),
            jax.ShapeDtypeStruct((C1, 1), jnp.float32),
        ],
    )(pT, q, x, y, W1, col(b1))

    h2_pre, s2, ss2 = pl.pallas_call(
        functools.partial(_bn_l2_kernel, count=count,
                          interp_dtype=interp_dtype),
        grid=grid,
        in_specs=[
            pl.BlockSpec((1, C1, BM), lambda b, j: (b, 0, j)),
            pl.BlockSpec((C1, 1), lambda b, j: (0, 0)),
            pl.BlockSpec((C1, 1), lambda b, j: (0, 0)),
            pl.BlockSpec((C1, 1), lambda b, j: (0, 0)),
            pl.BlockSpec((C1, 1), lambda b, j: (0, 0)),
            pl.BlockSpec((C2, C1), lambda b, j: (0, 0)),
            pl.BlockSpec((C2, 1), lambda b, j: (0, 0)),
        ],
        out_specs=[
            pl.BlockSpec((1, C2, BM), lambda b, j: (b, 0, j)),
            pl.BlockSpec((C2, 1), lambda b, j: (0, 0)),
            pl.BlockSpec((C2, 1), lambda b, j: (0, 0)),
        ],
        out_shape=[
            jax.ShapeDtypeStruct((B, C2, M), jnp.float32),
            jax.ShapeDtypeStruct((C2, 1), jnp.float32),
            jax.ShapeDtypeStruct((C2, 1), jnp.float32),
        ],
    )(h1_pre, s1, ss1, col(g1), col(be1), W2, col(b2))

    h = pl.pallas_call(
        functools.partial(_bn_out_kernel, count=count),
        grid=grid,
        in_specs=[
            pl.BlockSpec((1, C2, BM), lambda b, j: (b, 0, j)),
            pl.BlockSpec((C2, 1), lambda b, j: (0, 0)),
            pl.BlockSpec((C2, 1), lambda b, j: (0, 0)),
            pl.BlockSpec((C2, 1), lambda b, j: (0, 0)),
            pl.BlockSpec((C2, 1), lambda b, j: (0, 0)),
        ],
        out_specs=pl.BlockSpec((1, C2, BM), lambda b, j: (b, 0, j)),
        out_shape=jax.ShapeDtypeStruct((B, C2, M), jnp.float32),
    )(h2_pre, s2, ss2, col(g2), col(be2))

    return (q, h)
